# baseline (speedup 1.0000x reference)
import functools

import jax
import jax.numpy as jnp
from jax import lax
from jax.experimental import pallas as pl
from jax.experimental.pallas import tpu as pltpu
from jax.experimental.pallas import tpu_sc as plsc

BATCH = 16384
K = 1000
BR = 512
NB = BATCH // BR

L = 16
NC = 2
NS = 16
NW = NC * NS
CHUNK1 = BATCH // NS
CHUNK2 = BATCH // NW
KPAD = 1024
SW = 128
R1 = CHUNK1 // SW


NSCC = 4096
CW = NSCC // NW
NT = BATCH - NSCC
NBT = NT // BR
NQ = CW // L
JB = 8
CHK = (248, 248, 248, 256)
CHO = (0, 248, 496, 744)
CBUF = 256


def _sc_body(lbl_hbm, ct_hbm, cr_hbm, m_hbm, s_hbm, g_hbm, d_hbm,
             lbl1_v, ones_v, bins_v, bins_sh,
             lbl2_v, cr_v, d_v, labc_v, msg_v,
             cb0_v, cb1_v, sem0, sem1):
    cid = lax.axis_index("c")
    sid = lax.axis_index("s")
    wid = sid * NC + cid

    ones16 = jnp.ones((L,), jnp.float32)
    zeros16 = jnp.zeros((L,), jnp.float32)

    cbufs = [cb0_v, cb1_v]
    csems = [sem0, sem1]
    colb = wid * CW
    for t in range(2):
        pltpu.async_copy(
            ct_hbm.at[pl.ds(CHO[t], CHK[t]), pl.ds(NT + colb, CW)],
            cbufs[t].at[pl.ds(0, CHK[t])], csems[t])
    pltpu.sync_copy(lbl_hbm.at[pl.ds(NT + colb, CW)], labc_v)

    base2 = wid * CHUNK2
    pltpu.sync_copy(lbl_hbm.at[pl.ds(base2, CHUNK2)], lbl2_v)

    def _fill(j, carry):
        bins_v[pl.ds(j * L, L)] = zeros16
        return carry
    lax.fori_loop(0, KPAD // L, _fill, 0)

    def _fill1(j, carry):
        ones_v[pl.ds(j * L, L)] = ones16
        return carry
    lax.fori_loop(0, SW // L, _fill1, 0)

    base1 = sid * CHUNK1
    for j in range(R1):
        pltpu.sync_copy(lbl_hbm.at[pl.ds(base1 + j * SW, SW)], lbl1_v.at[j])

    @pl.when(sid == 0)
    def _():
        pltpu.sync_copy(bins_v, bins_sh)

    plsc.subcore_barrier()
    for j in range(R1):
        pltpu.sync_copy(ones_v, bins_sh.at[lbl1_v.at[j]], add=True)
    plsc.subcore_barrier()

    pltpu.sync_copy(bins_sh, bins_v)

    for t in range(CHUNK2 // SW):
        pltpu.sync_copy(
            bins_sh.at[lbl2_v.at[pl.ds(t * SW, SW)]],
            cr_v.at[pl.ds(t * SW, SW)],
        )
    pltpu.sync_copy(cr_v, cr_hbm.at[pl.ds(base2, CHUNK2)])

    @pl.when((cid == 0) & (sid == 0))
    def _():
        def _dd(j, a):
            return a + jnp.where(bins_v[pl.ds(j * L, L)] > 0.0, 1.0, 0.0)
        d_v[...] = lax.fori_loop(0, KPAD // L, _dd, zeros16)
        pltpu.sync_copy(d_v, d_hbm)

    neg16 = jnp.full((L,), -1e30, jnp.float32)
    labq = [labc_v[pl.ds(q * L, L)] for q in range(NQ)]
    m8 = [neg16] * NQ
    s8 = [zeros16] * NQ
    g8 = [zeros16] * NQ

    for tc in range(4):
        buf = cbufs[tc % 2]
        rows = CHK[tc]
        pltpu.make_async_copy(
            ct_hbm.at[pl.ds(0, rows), pl.ds(0, CW)],
            buf.at[pl.ds(0, rows)], csems[tc % 2]).wait()

        for half in range(2):
            qs = list(range(4 * half, 4 * half + 4))

            def _mx(t, mc, _qs=qs, _buf=buf):
                mc = list(mc)
                for u in range(JB):
                    row = t * JB + u
                    for i, q in enumerate(_qs):
                        mc[i] = jnp.maximum(mc[i], _buf[row, pl.ds(q * L, L)])
                return tuple(mc)
            mc4 = lax.fori_loop(0, rows // JB, _mx, (neg16,) * 4)

            mn4 = [jnp.maximum(m8[q], mc4[i]) for i, q in enumerate(qs)]
            sc4 = [s8[q] * jnp.exp(m8[q] - mn4[i]) for i, q in enumerate(qs)]

            def _sg(t, sg, _qs=qs, _buf=buf, _mn=mn4, _off=CHO[tc]):
                s, g = [list(x) for x in sg]
                for u in range(JB):
                    row = t * JB + u
                    for i, q in enumerate(_qs):
                        x = _buf[row, pl.ds(q * L, L)]
                        s[i] = s[i] + jnp.exp(x - _mn[i])
                        g[i] = g[i] + jnp.where(labq[q] == _off + row, x, zeros16)
                return tuple(s), tuple(g)
            s4, g4 = lax.fori_loop(0, rows // JB, _sg,
                                   (tuple(sc4), tuple(g8[q] for q in qs)))

            for i, q in enumerate(qs):
                m8[q] = mn4[i]
                s8[q] = s4[i]
                g8[q] = g4[i]

        if tc + 2 < 4:
            pltpu.async_copy(
                ct_hbm.at[pl.ds(CHO[tc + 2], CHK[tc + 2]), pl.ds(NT + colb, CW)],
                cbufs[tc % 2].at[pl.ds(0, CHK[tc + 2])], csems[tc % 2])

    for q in range(NQ):
        msg_v[pl.ds(q * L, L)] = m8[q]
        msg_v[pl.ds(CW + q * L, L)] = s8[q]
        msg_v[pl.ds(2 * CW + q * L, L)] = g8[q]

    pltpu.sync_copy(msg_v.at[pl.ds(0, CW)], m_hbm.at[pl.ds(colb, CW)])
    pltpu.sync_copy(msg_v.at[pl.ds(CW, CW)], s_hbm.at[pl.ds(colb, CW)])
    pltpu.sync_copy(msg_v.at[pl.ds(2 * CW, CW)], g_hbm.at[pl.ds(colb, CW)])


_sc_stats = functools.partial(
    pl.kernel,
    mesh=plsc.VectorSubcoreMesh(core_axis_name="c", subcore_axis_name="s"),
    out_type=[
        jax.ShapeDtypeStruct((BATCH,), jnp.float32),
        jax.ShapeDtypeStruct((NSCC,), jnp.float32),
        jax.ShapeDtypeStruct((NSCC,), jnp.float32),
        jax.ShapeDtypeStruct((NSCC,), jnp.float32),
        jax.ShapeDtypeStruct((L,), jnp.float32),
    ],
    scratch_types=[
        pltpu.VMEM((R1, SW), jnp.int32),
        pltpu.VMEM((SW,), jnp.float32),
        pltpu.VMEM((KPAD,), jnp.float32),
        pltpu.VMEM_SHARED((KPAD,), jnp.float32),
        pltpu.VMEM((CHUNK2,), jnp.int32),
        pltpu.VMEM((CHUNK2,), jnp.float32),
        pltpu.VMEM((L,), jnp.float32),
        pltpu.VMEM((CW,), jnp.int32),
        pltpu.VMEM((3 * CW,), jnp.float32),
        pltpu.VMEM((CBUF, CW), jnp.float32),
        pltpu.VMEM((CBUF, CW), jnp.float32),
        pltpu.SemaphoreType.DMA,
        pltpu.SemaphoreType.DMA,
    ],
)(_sc_body)


def _tc_body(lbl_ref, ct_ref, nll_ref):
    cb = ct_ref[...]
    m = jnp.max(cb, axis=0, keepdims=True)
    s = jnp.sum(jnp.exp(cb - m), axis=0, keepdims=True)
    lse = m + jnp.log(s)

    onehot = jax.lax.broadcasted_iota(jnp.int32, (K, BR), 0) == lbl_ref[...]
    g = jnp.sum(jnp.where(onehot, cb, 0.0), axis=0, keepdims=True)
    nll_ref[...] = lse - g


def _fin_body(nll_ref, cr_ref, m_ref, s_ref, g_ref, d_ref, loss_ref):
    cr = cr_ref[...]
    t1 = jnp.sum(nll_ref[...] / cr[:, :NT], keepdims=True)
    nll_sc = m_ref[...] + jnp.log(s_ref[...]) - g_ref[...]
    t2 = jnp.sum(nll_sc / cr[:, NT:], keepdims=True)
    loss_ref[...] = (t1 + t2) / jnp.sum(d_ref[...], keepdims=True)


def kernel(c, pseudo_label):
    lbl = pseudo_label.astype(jnp.int32)
    ct = c.T
    cr, m_sc, s_sc, g_sc, dv = _sc_stats(lbl, ct)

    nll = pl.pallas_call(
        _tc_body,
        grid=(NBT,),
        in_specs=[
            pl.BlockSpec((1, BR), lambda k: (0, k)),
            pl.BlockSpec((K, BR), lambda k: (0, k)),
        ],
        out_specs=pl.BlockSpec((1, BR), lambda k: (0, k)),
        out_shape=jax.ShapeDtypeStruct((1, NT), jnp.float32),
    )(lbl[:NT].reshape(1, NT), ct)

    loss = pl.pallas_call(
        _fin_body,
        in_specs=[
            pl.BlockSpec((1, NT), lambda: (0, 0)),
            pl.BlockSpec((1, BATCH), lambda: (0, 0)),
            pl.BlockSpec((1, NSCC), lambda: (0, 0)),
            pl.BlockSpec((1, NSCC), lambda: (0, 0)),
            pl.BlockSpec((1, NSCC), lambda: (0, 0)),
            pl.BlockSpec((1, L), lambda: (0, 0)),
        ],
        out_specs=pl.BlockSpec((1, 1), lambda: (0, 0)),
        out_shape=jax.ShapeDtypeStruct((1, 1), jnp.float32),
    )(nll, cr.reshape(1, BATCH), m_sc.reshape(1, NSCC),
      s_sc.reshape(1, NSCC), g_sc.reshape(1, NSCC), dv.reshape(1, L))
    return loss[0, 0]


# speedup vs baseline: 1.2594x; 1.2594x over previous
import functools

import jax
import jax.numpy as jnp
from jax import lax
from jax.experimental import pallas as pl
from jax.experimental.pallas import tpu as pltpu
from jax.experimental.pallas import tpu_sc as plsc

BATCH = 16384
K = 1000
BR = 512
NB = BATCH // BR

L = 16
NC = 2
NS = 16
NW = NC * NS
CHUNK1 = BATCH // NS
CHUNK2 = BATCH // NW
KPAD = 1024
SW = 128
R1 = CHUNK1 // SW


def _sc_body(lbl_hbm, cr_hbm, d_hbm,
             lbl1_v, ones_v, bins_v, bins_sh,
             lbl2_v, cr_v, d_v):
    cid = lax.axis_index("c")
    sid = lax.axis_index("s")
    wid = sid * NC + cid

    ones16 = jnp.ones((L,), jnp.float32)
    zeros16 = jnp.zeros((L,), jnp.float32)

    base2 = wid * CHUNK2
    pltpu.sync_copy(lbl_hbm.at[pl.ds(base2, CHUNK2)], lbl2_v)

    def _fill(j, carry):
        bins_v[pl.ds(j * L, L)] = zeros16
        return carry
    lax.fori_loop(0, KPAD // L, _fill, 0)

    def _fill1(j, carry):
        ones_v[pl.ds(j * L, L)] = ones16
        return carry
    lax.fori_loop(0, SW // L, _fill1, 0)

    base1 = sid * CHUNK1
    for j in range(R1):
        pltpu.sync_copy(lbl_hbm.at[pl.ds(base1 + j * SW, SW)], lbl1_v.at[j])

    @pl.when(sid == 0)
    def _():
        pltpu.sync_copy(bins_v, bins_sh)

    plsc.subcore_barrier()
    for j in range(R1):
        pltpu.sync_copy(ones_v, bins_sh.at[lbl1_v.at[j]], add=True)
    plsc.subcore_barrier()

    pltpu.sync_copy(bins_sh, bins_v)

    for t in range(CHUNK2 // SW):
        pltpu.sync_copy(
            bins_sh.at[lbl2_v.at[pl.ds(t * SW, SW)]],
            cr_v.at[pl.ds(t * SW, SW)],
        )
    pltpu.sync_copy(cr_v, cr_hbm.at[pl.ds(base2, CHUNK2)])

    @pl.when((cid == 0) & (sid == 0))
    def _():
        def _dd(j, a):
            return a + jnp.where(bins_v[pl.ds(j * L, L)] > 0.0, 1.0, 0.0)
        d_v[...] = lax.fori_loop(0, KPAD // L, _dd, zeros16)
        pltpu.sync_copy(d_v, d_hbm)


_sc_stats = functools.partial(
    pl.kernel,
    mesh=plsc.VectorSubcoreMesh(core_axis_name="c", subcore_axis_name="s"),
    out_type=[
        jax.ShapeDtypeStruct((BATCH,), jnp.float32),
        jax.ShapeDtypeStruct((L,), jnp.float32),
    ],
    scratch_types=[
        pltpu.VMEM((R1, SW), jnp.int32),
        pltpu.VMEM((SW,), jnp.float32),
        pltpu.VMEM((KPAD,), jnp.float32),
        pltpu.VMEM_SHARED((KPAD,), jnp.float32),
        pltpu.VMEM((CHUNK2,), jnp.int32),
        pltpu.VMEM((CHUNK2,), jnp.float32),
        pltpu.VMEM((L,), jnp.float32),
    ],
)(_sc_body)


def _tc_body(lbl_ref, ct_ref, nll_ref):
    cb = ct_ref[...]
    m = jnp.max(cb, axis=0, keepdims=True)
    s = jnp.sum(jnp.exp(cb - m), axis=0, keepdims=True)
    lse = m + jnp.log(s)

    onehot = jax.lax.broadcasted_iota(jnp.int32, (K, BR), 0) == lbl_ref[...]
    g = jnp.sum(jnp.where(onehot, cb, 0.0), axis=0, keepdims=True)
    nll_ref[...] = lse - g


def _fin_body(nll_ref, cr_ref, d_ref, loss_ref):
    t = jnp.sum(nll_ref[...] / cr_ref[...], keepdims=True)
    loss_ref[...] = t / jnp.sum(d_ref[...], keepdims=True)


def kernel(c, pseudo_label):
    lbl = pseudo_label.astype(jnp.int32)

    nll = pl.pallas_call(
        _tc_body,
        grid=(NB,),
        in_specs=[
            pl.BlockSpec((1, BR), lambda k: (0, k)),
            pl.BlockSpec((K, BR), lambda k: (0, k)),
        ],
        out_specs=pl.BlockSpec((1, BR), lambda k: (0, k)),
        out_shape=jax.ShapeDtypeStruct((1, BATCH), jnp.float32),
    )(lbl.reshape(1, BATCH), c.T)

    cr, dv = _sc_stats(lbl)

    loss = pl.pallas_call(
        _fin_body,
        in_specs=[
            pl.BlockSpec((1, BATCH), lambda: (0, 0)),
            pl.BlockSpec((1, BATCH), lambda: (0, 0)),
            pl.BlockSpec((1, L), lambda: (0, 0)),
        ],
        out_specs=pl.BlockSpec((1, 1), lambda: (0, 0)),
        out_shape=jax.ShapeDtypeStruct((1, 1), jnp.float32),
    )(nll, cr.reshape(1, BATCH), dv.reshape(1, L))
    return loss[0, 0]
